# baseline (speedup 1.0000x reference)
import functools

import jax
import jax.numpy as jnp
from jax import lax
from jax.experimental import pallas as pl
from jax.experimental.pallas import tpu as pltpu
from jax.experimental.pallas import tpu_sc as plsc

_NC = 2
_NS = 16
_NW = _NC * _NS
_CHUNK = 128
_K = 16
_W = 16


def _sc_gather_planar(jobs):
    ch = 256
    maxn = max(j[2] for j in jobs)
    maxrpw = max(j[1].shape[0] // _NW for j in jobs)
    mesh = plsc.VectorSubcoreMesh(core_axis_name="c", subcore_axis_name="s")
    out_type = tuple(
        jax.ShapeDtypeStruct((j[1].shape[0], 16), jnp.float32) for j in jobs
    )
    scratch = (
        [pltpu.VMEM((maxn,), jnp.float32) for _ in range(3)]
        + [pltpu.VMEM((maxrpw,), jnp.int32)]
        + [pltpu.VMEM((ch, 16), jnp.float32) for _ in range(2)]
        + [pltpu.SemaphoreType.DMA]
    )
    nin = 4 * len(jobs) + 1

    def body(*refs):
        ins = refs[:nin]
        outs_hbm = refs[nin : nin + len(jobs)]
        plane_v = refs[nin + len(jobs) : nin + len(jobs) + 3]
        idx_v = refs[nin + len(jobs) + 3]
        fbufs = refs[nin + len(jobs) + 4 : nin + len(jobs) + 6]
        sem = refs[-1]
        zeros_hbm = ins[4 * len(jobs)]
        wid = lax.axis_index("s") * _NC + lax.axis_index("c")
        iota16 = lax.iota(jnp.int32, 16)
        cols = [jnp.full((16,), p, jnp.int32) for p in range(3)]
        for fb in fbufs:
            pltpu.sync_copy(zeros_hbm, fb)

        for ij, (planes, idx, n, bm, m) in enumerate(jobs):
            plane_hbm = ins[4 * ij : 4 * ij + 3]
            idx_hbm = ins[4 * ij + 3]
            out_hbm = outs_hbm[ij]
            rpw = idx.shape[0] // _NW
            nst = rpw // ch
            base = pl.multiple_of(wid * rpw, ch)
            batch = lax.rem(base, bm) // m
            same_idx = ij > 0 and jobs[ij - 1][1] is idx
            stage = []
            if not same_idx:
                stage.append(pltpu.async_copy(
                    idx_hbm.at[pl.ds(base, rpw)],
                    idx_v.at[pl.ds(0, rpw)], sem))
            for p in range(3):
                stage.append(pltpu.async_copy(
                    plane_hbm[p].at[pl.ds(pl.multiple_of(batch * n, 8), n)],
                    plane_v[p].at[pl.ds(0, n)], sem))
            for cp in stage:
                cp.wait()

            def fill(s, buf):
                for g in range(ch // 16):
                    iv = idx_v[pl.ds(s * ch + g * 16, 16)]
                    rows = iota16 + (g * 16)
                    for p in range(3):
                        vals = plsc.load_gather(plane_v[p], [iv])
                        plsc.store_scatter(buf, [rows, cols[p]], vals)

            cps = [None, None]
            for s in range(nst):
                buf = fbufs[s % 2]
                if cps[s % 2] is not None:
                    cps[s % 2].wait()
                fill(s, buf)
                cps[s % 2] = pltpu.async_copy(
                    buf, out_hbm.at[pl.ds(base + s * ch, ch)], sem
                )
            for cp in cps:
                if cp is not None:
                    cp.wait()

    args = []
    for planes, idx, n, bm, m in jobs:
        args += list(planes) + [idx]
    args.append(jnp.zeros((ch, 16), jnp.float32))
    fn = pl.kernel(
        body, out_type=out_type, mesh=mesh, scratch_types=scratch,
        compiler_params=pltpu.CompilerParams(needs_layout_passes=False),
    )
    return list(fn(*args))


def _sc_gather(tables, idx):
    nt = len(tables)
    assert nt == 1
    table = tables[0]
    d = table.shape[1]
    r = idx.shape[0]
    rpw = r // _NW
    chunk = min(_CHUNK, 16384 // d)
    nch = rpw // chunk
    mesh = plsc.VectorSubcoreMesh(core_axis_name="c", subcore_axis_name="s")
    out_type = jax.ShapeDtypeStruct((r, d), jnp.float32)
    scratch = (
        [pltpu.VMEM((nch, chunk), jnp.int32)]
        + [pltpu.VMEM((chunk, d), jnp.float32) for _ in range(2)]
        + [pltpu.SemaphoreType.DMA, pltpu.SemaphoreType.DMA]
    )

    def body(tab, idx_hbm, out_hbm, idx_v, buf0, buf1, gsem, osem):
        bufs = (buf0, buf1)
        wid = lax.axis_index("s") * _NC + lax.axis_index("c")
        base = pl.multiple_of(wid * rpw, chunk * 8)
        pltpu.sync_copy(
            idx_hbm.at[pl.ds(pl.multiple_of(base // chunk, 8), nch)], idx_v
        )
        gcp = [None, None]
        ocp = [None, None]

        def out_copy(c):
            return pltpu.async_copy(
                bufs[c % 2],
                out_hbm.at[pl.ds(base + c * chunk, chunk)],
                osem,
            )

        for c in range(nch):
            b = c % 2
            if ocp[b] is not None:
                ocp[b].wait()
                ocp[b] = None
            gcp[b] = pltpu.async_copy(tab.at[idx_v.at[c]], bufs[b], gsem)
            if c > 0:
                pb = 1 - b
                gcp[pb].wait()
                ocp[pb] = out_copy(c - 1)
        lb = (nch - 1) % 2
        gcp[lb].wait()
        ocp[lb] = out_copy(nch - 1)
        for cp in ocp:
            if cp is not None:
                cp.wait()

    fn = pl.kernel(body, out_type=out_type, mesh=mesh, scratch_types=scratch)
    return [fn(table, idx.reshape(r // chunk, chunk))]


def _weightnet(gx_ref, sx, wnwp, wnb):
    wks = []
    for k in range(_K):
        diff = gx_ref[k] - sx
        wk = jnp.dot(diff, wnwp, preferred_element_type=jnp.float32)
        wks.append(jnp.maximum(wk + wnb, 0.0))
    return wks


def _tc_layer0(g_xyz, g_feat, sxyz_p, wnwp, wn_b, lin_w3, lin_b, mt):
    bm = sxyz_p.shape[0]

    def body(gx_ref, gf_ref, sx_ref, wnw_ref, wnb_ref, w3_ref, b_ref, o_ref):
        wks = _weightnet(gx_ref, sx_ref[...], wnw_ref[...], wnb_ref[...])
        out = None
        for c in range(3):
            acc = None
            for k in range(_K):
                t = wks[k] * gf_ref[k][:, c : c + 1]
                acc = t if acc is None else acc + t
            part = jnp.dot(acc, w3_ref[c], preferred_element_type=jnp.float32)
            out = part if out is None else out + part
        o_ref[...] = jnp.maximum(out + b_ref[...], 0.0)

    return pl.pallas_call(
        body,
        grid=(bm // mt,),
        in_specs=[
            pl.BlockSpec((_K, mt, 16), lambda i: (0, i, 0)),
            pl.BlockSpec((_K, mt, 16), lambda i: (0, i, 0)),
            pl.BlockSpec((mt, 16), lambda i: (i, 0)),
            pl.BlockSpec((16, 16), lambda i: (0, 0)),
            pl.BlockSpec((1, 16), lambda i: (0, 0)),
            pl.BlockSpec((3, 16, 256), lambda i: (0, 0, 0)),
            pl.BlockSpec((1, 256), lambda i: (0, 0)),
        ],
        out_specs=pl.BlockSpec((mt, 256), lambda i: (i, 0)),
        out_shape=jax.ShapeDtypeStruct((bm, 256), jnp.float32),
        compiler_params=pltpu.CompilerParams(
            dimension_semantics=("arbitrary",)
        ),
    )(g_xyz, g_feat, sxyz_p, wnwp, wn_b.reshape(1, 16), lin_w3,
      lin_b.reshape(1, 256))


def _tc_layer1(g_xyz, g_feat, sxyz_p, wnwp, wn_b, w3_bf, lin_b, mt):
    bm = sxyz_p.shape[0]

    def body(gx_ref, gf_ref, sx_ref, wnw_ref, wnb_ref, w3_ref, b_ref, o_ref):
        wks = _weightnet(gx_ref, sx_ref[...], wnw_ref[...], wnb_ref[...])
        wks = [wk.astype(jnp.bfloat16) for wk in wks]
        gfs = [gf_ref[k].astype(jnp.bfloat16) for k in range(_K)]
        out = None
        for w in range(_W):
            acc = None
            for k in range(_K):
                t = gfs[k] * wks[k][:, w : w + 1]
                acc = t if acc is None else acc + t
            part = jnp.dot(acc, w3_ref[w], preferred_element_type=jnp.float32)
            out = part if out is None else out + part
        o_ref[...] = jnp.maximum(out + b_ref[...], 0.0)

    return pl.pallas_call(
        body,
        grid=(bm // mt,),
        in_specs=[
            pl.BlockSpec((_K, mt, 16), lambda i: (0, i, 0)),
            pl.BlockSpec((_K, mt, 256), lambda i: (0, i, 0)),
            pl.BlockSpec((mt, 16), lambda i: (i, 0)),
            pl.BlockSpec((16, 16), lambda i: (0, 0)),
            pl.BlockSpec((1, 16), lambda i: (0, 0)),
            pl.BlockSpec((_W, 256, 1024), lambda i: (0, 0, 0)),
            pl.BlockSpec((1, 1024), lambda i: (0, 0)),
        ],
        out_specs=pl.BlockSpec((mt, 1024), lambda i: (i, 0)),
        out_shape=jax.ShapeDtypeStruct((bm, 1024), jnp.float32),
        compiler_params=pltpu.CompilerParams(
            dimension_semantics=("arbitrary",)
        ),
    )(g_xyz, g_feat, sxyz_p, wnwp, wn_b.reshape(1, 16), w3_bf,
      lin_b.reshape(1, 1024))


def _pad16(x3):
    n = x3.shape[0]
    return jnp.concatenate(
        [x3, jnp.zeros((n, 13), dtype=x3.dtype)], axis=1
    )


def _flat_idx(nei_inds, n):
    b = nei_inds.shape[0]
    off = (jnp.arange(b, dtype=jnp.int32) * n)[:, None, None]
    return (nei_inds + off).transpose(2, 0, 1).reshape(-1)


def _wmajor(lin_w, cin):
    cout = lin_w.shape[1]
    return lin_w.reshape(cin, _W, cout).transpose(1, 0, 2).reshape(_W * cin, cout)


def kernel(xyz0, xyz1, xyz2, init_feats, nei_inds0, nei_inds1,
           inv_neighbors0, inv_neighbors1, inv_k0, inv_k1, inv_idx0, inv_idx1,
           wn0_W, wn0_b, lin0_W, lin0_b, wn1_W, wn1_b, lin1_W, lin1_b):
    b, n0, _ = xyz0.shape
    m0 = xyz1.shape[1]
    m1 = xyz2.shape[1]

    idx0l = nei_inds0.transpose(2, 0, 1).reshape(-1)
    idx1l = nei_inds1.transpose(2, 0, 1).reshape(-1)
    gx0, gf0 = _sc_gather_planar([
        ([xyz0[..., d].reshape(-1) for d in range(3)], idx0l, n0, b * m0, m0),
        ([init_feats[..., d].reshape(-1) for d in range(3)], idx0l, n0,
         b * m0, m0),
    ])
    (gx1,) = _sc_gather_planar([
        ([xyz1[..., d].reshape(-1) for d in range(3)], idx1l, m0, b * m1, m1),
    ])
    gx0 = gx0.reshape(_K, b * m0, 16)
    gf0 = gf0.reshape(_K, b * m0, 16)
    wnw0p = jnp.concatenate(
        [wn0_W, jnp.zeros((13, 16), dtype=jnp.float32)], axis=0
    )
    f1 = _tc_layer0(
        gx0, gf0, _pad16(xyz1.reshape(b * m0, 3)), wnw0p, wn0_b,
        lin0_W.reshape(3, 16, 256), lin0_b, mt=512,
    )

    idx1 = _flat_idx(nei_inds1, m0)
    (gf1,) = _sc_gather([f1], idx1)
    gx1 = gx1.reshape(_K, b * m1, 16)
    gf1 = gf1.reshape(_K, b * m1, 256)
    wnw1p = jnp.concatenate(
        [wn1_W, jnp.zeros((13, 16), dtype=jnp.float32)], axis=0
    )
    w3_bf = _wmajor(lin1_W, 256).astype(jnp.bfloat16).reshape(_W, 256, 1024)
    out = _tc_layer1(
        gx1, gf1, _pad16(xyz2.reshape(b * m1, 3)), wnw1p, wn1_b,
        w3_bf, lin1_b, mt=128,
    )
    return out.reshape(b, m1, 1024)


# speedup vs baseline: 1.0192x; 1.0192x over previous
import functools

import jax
import jax.numpy as jnp
from jax import lax
from jax.experimental import pallas as pl
from jax.experimental.pallas import tpu as pltpu
from jax.experimental.pallas import tpu_sc as plsc

_NC = 2
_NS = 16
_NW = _NC * _NS
_CHUNK = 128
_K = 16
_W = 16


def _sc_gather_planar(jobs):
    ch = 256
    maxn = max(j[2] for j in jobs)
    maxrpw = max(j[1].shape[0] // _NW for j in jobs)
    mesh = plsc.VectorSubcoreMesh(core_axis_name="c", subcore_axis_name="s")
    out_type = tuple(
        jax.ShapeDtypeStruct((j[1].shape[0], 16), jnp.float32) for j in jobs
    )
    scratch = (
        [pltpu.VMEM((maxn,), jnp.float32) for _ in range(3)]
        + [pltpu.VMEM((maxrpw,), jnp.int32)]
        + [pltpu.VMEM((ch, 16), jnp.float32) for _ in range(2)]
        + [pltpu.SemaphoreType.DMA]
    )
    nin = 4 * len(jobs) + 1

    def body(*refs):
        ins = refs[:nin]
        outs_hbm = refs[nin : nin + len(jobs)]
        plane_v = refs[nin + len(jobs) : nin + len(jobs) + 3]
        idx_v = refs[nin + len(jobs) + 3]
        fbufs = refs[nin + len(jobs) + 4 : nin + len(jobs) + 6]
        sem = refs[-1]
        zeros_hbm = ins[4 * len(jobs)]
        wid = lax.axis_index("s") * _NC + lax.axis_index("c")
        iota16 = lax.iota(jnp.int32, 16)
        cols = [jnp.full((16,), p, jnp.int32) for p in range(3)]
        for fb in fbufs:
            pltpu.sync_copy(zeros_hbm, fb)

        for ij, (planes, idx, n, bm, m) in enumerate(jobs):
            plane_hbm = ins[4 * ij : 4 * ij + 3]
            idx_hbm = ins[4 * ij + 3]
            out_hbm = outs_hbm[ij]
            rpw = idx.shape[0] // _NW
            nst = rpw // ch
            base = pl.multiple_of(wid * rpw, ch)
            batch = lax.rem(base, bm) // m
            same_idx = ij > 0 and jobs[ij - 1][1] is idx
            stage = []
            if not same_idx:
                stage.append(pltpu.async_copy(
                    idx_hbm.at[pl.ds(base, rpw)],
                    idx_v.at[pl.ds(0, rpw)], sem))
            for p in range(3):
                stage.append(pltpu.async_copy(
                    plane_hbm[p].at[pl.ds(pl.multiple_of(batch * n, 8), n)],
                    plane_v[p].at[pl.ds(0, n)], sem))
            for cp in stage:
                cp.wait()

            def fill(s, buf):
                for g in range(ch // 16):
                    iv = idx_v[pl.ds(s * ch + g * 16, 16)]
                    rows = iota16 + (g * 16)
                    for p in range(3):
                        vals = plsc.load_gather(plane_v[p], [iv])
                        plsc.store_scatter(buf, [rows, cols[p]], vals)

            cps = [None, None]
            for s in range(nst):
                buf = fbufs[s % 2]
                if cps[s % 2] is not None:
                    cps[s % 2].wait()
                fill(s, buf)
                cps[s % 2] = pltpu.async_copy(
                    buf, out_hbm.at[pl.ds(base + s * ch, ch)], sem
                )
            for cp in cps:
                if cp is not None:
                    cp.wait()

    args = []
    for planes, idx, n, bm, m in jobs:
        args += list(planes) + [idx]
    args.append(jnp.zeros((ch, 16), jnp.float32))
    fn = pl.kernel(
        body, out_type=out_type, mesh=mesh, scratch_types=scratch,
        compiler_params=pltpu.CompilerParams(needs_layout_passes=False),
    )
    return list(fn(*args))


def _sc_gather(tables, idx):
    nt = len(tables)
    assert nt == 1
    table = tables[0]
    d = table.shape[1]
    r = idx.shape[0]
    rpw = r // _NW
    chunk = min(_CHUNK, 16384 // d)
    nch = rpw // chunk
    mesh = plsc.VectorSubcoreMesh(core_axis_name="c", subcore_axis_name="s")
    out_type = jax.ShapeDtypeStruct((r, d), jnp.float32)
    scratch = (
        [pltpu.VMEM((nch, chunk), jnp.int32)]
        + [pltpu.VMEM((chunk, d), jnp.float32) for _ in range(2)]
        + [pltpu.SemaphoreType.DMA, pltpu.SemaphoreType.DMA]
    )

    def body(tab, idx_hbm, out_hbm, idx_v, buf0, buf1, gsem, osem):
        bufs = (buf0, buf1)
        wid = lax.axis_index("s") * _NC + lax.axis_index("c")
        base = pl.multiple_of(wid * rpw, chunk * 8)
        pltpu.sync_copy(
            idx_hbm.at[pl.ds(pl.multiple_of(base // chunk, 8), nch)], idx_v
        )
        gcp = [None, None]
        ocp = [None, None]

        def out_copy(c):
            return pltpu.async_copy(
                bufs[c % 2],
                out_hbm.at[pl.ds(base + c * chunk, chunk)],
                osem,
            )

        for c in range(nch):
            b = c % 2
            if ocp[b] is not None:
                ocp[b].wait()
                ocp[b] = None
            gcp[b] = pltpu.async_copy(tab.at[idx_v.at[c]], bufs[b], gsem)
            if c > 0:
                pb = 1 - b
                gcp[pb].wait()
                ocp[pb] = out_copy(c - 1)
        lb = (nch - 1) % 2
        gcp[lb].wait()
        ocp[lb] = out_copy(nch - 1)
        for cp in ocp:
            if cp is not None:
                cp.wait()

    fn = pl.kernel(body, out_type=out_type, mesh=mesh, scratch_types=scratch)
    return [fn(table, idx.reshape(r // chunk, chunk))]


def _weightnet(gx_ref, sx, wnwp, wnb):
    wks = []
    for k in range(_K):
        diff = gx_ref[k] - sx
        wk = jnp.dot(diff, wnwp, preferred_element_type=jnp.float32)
        wks.append(jnp.maximum(wk + wnb, 0.0))
    return wks


def _tc_layer0(g_xyz, g_feat, sxyz_p, wnwp, wn_b, lin_w3, lin_b, mt):
    bm = sxyz_p.shape[0]

    def body(gx_ref, gf_ref, sx_ref, wnw_ref, wnb_ref, w3_ref, b_ref, o_ref):
        wks = _weightnet(gx_ref, sx_ref[...], wnw_ref[...], wnb_ref[...])
        out = None
        for c in range(3):
            acc = None
            for k in range(_K):
                t = wks[k] * gf_ref[k][:, c : c + 1]
                acc = t if acc is None else acc + t
            part = jnp.dot(acc, w3_ref[c], preferred_element_type=jnp.float32)
            out = part if out is None else out + part
        o_ref[...] = jnp.maximum(out + b_ref[...], 0.0)

    return pl.pallas_call(
        body,
        grid=(bm // mt,),
        in_specs=[
            pl.BlockSpec((_K, mt, 16), lambda i: (0, i, 0)),
            pl.BlockSpec((_K, mt, 16), lambda i: (0, i, 0)),
            pl.BlockSpec((mt, 16), lambda i: (i, 0)),
            pl.BlockSpec((16, 16), lambda i: (0, 0)),
            pl.BlockSpec((1, 16), lambda i: (0, 0)),
            pl.BlockSpec((3, 16, 256), lambda i: (0, 0, 0)),
            pl.BlockSpec((1, 256), lambda i: (0, 0)),
        ],
        out_specs=pl.BlockSpec((mt, 256), lambda i: (i, 0)),
        out_shape=jax.ShapeDtypeStruct((bm, 256), jnp.float32),
        compiler_params=pltpu.CompilerParams(
            dimension_semantics=("arbitrary",)
        ),
    )(g_xyz, g_feat, sxyz_p, wnwp, wn_b.reshape(1, 16), lin_w3,
      lin_b.reshape(1, 256))


def _tc_layer1(g_xyz, g_feat, sxyz_p, wnwp, wn_b, w3_bf, lin_b, mt):
    bm = sxyz_p.shape[0]

    def body(gx_ref, gf_ref, sx_ref, wnw_ref, wnb_ref, w3_ref, b_ref, o_ref):
        wks = _weightnet(gx_ref, sx_ref[...], wnw_ref[...], wnb_ref[...])
        wks = [wk.astype(jnp.bfloat16) for wk in wks]
        gfs = [gf_ref[k].astype(jnp.bfloat16) for k in range(_K)]
        out = None
        for w in range(_W):
            acc = None
            for k in range(_K):
                t = gfs[k] * wks[k][:, w : w + 1]
                acc = t if acc is None else acc + t
            part = jnp.dot(acc, w3_ref[w], preferred_element_type=jnp.float32)
            out = part if out is None else out + part
        o_ref[...] = jnp.maximum(out + b_ref[...], 0.0)

    return pl.pallas_call(
        body,
        grid=(bm // mt,),
        in_specs=[
            pl.BlockSpec((_K, mt, 16), lambda i: (0, i, 0)),
            pl.BlockSpec((_K, mt, 256), lambda i: (0, i, 0)),
            pl.BlockSpec((mt, 16), lambda i: (i, 0)),
            pl.BlockSpec((16, 16), lambda i: (0, 0)),
            pl.BlockSpec((1, 16), lambda i: (0, 0)),
            pl.BlockSpec((_W, 256, 1024), lambda i: (0, 0, 0)),
            pl.BlockSpec((1, 1024), lambda i: (0, 0)),
        ],
        out_specs=pl.BlockSpec((mt, 1024), lambda i: (i, 0)),
        out_shape=jax.ShapeDtypeStruct((bm, 1024), jnp.float32),
        compiler_params=pltpu.CompilerParams(
            dimension_semantics=("arbitrary",)
        ),
    )(g_xyz, g_feat, sxyz_p, wnwp, wn_b.reshape(1, 16), w3_bf,
      lin_b.reshape(1, 1024))


def _pad16(x3):
    n = x3.shape[0]
    return jnp.concatenate(
        [x3, jnp.zeros((n, 13), dtype=x3.dtype)], axis=1
    )


def _flat_idx(nei_inds, n):
    b = nei_inds.shape[0]
    off = (jnp.arange(b, dtype=jnp.int32) * n)[:, None, None]
    return (nei_inds + off).transpose(2, 0, 1).reshape(-1)


def _wmajor(lin_w, cin):
    cout = lin_w.shape[1]
    return lin_w.reshape(cin, _W, cout).transpose(1, 0, 2).reshape(_W * cin, cout)


def kernel(xyz0, xyz1, xyz2, init_feats, nei_inds0, nei_inds1,
           inv_neighbors0, inv_neighbors1, inv_k0, inv_k1, inv_idx0, inv_idx1,
           wn0_W, wn0_b, lin0_W, lin0_b, wn1_W, wn1_b, lin1_W, lin1_b):
    b, n0, _ = xyz0.shape
    m0 = xyz1.shape[1]
    m1 = xyz2.shape[1]

    idx0l = nei_inds0.transpose(2, 0, 1).reshape(-1)
    idx1l = nei_inds1.transpose(2, 0, 1).reshape(-1)
    gx0, gf0, gx1 = _sc_gather_planar([
        ([xyz0[..., d].reshape(-1) for d in range(3)], idx0l, n0, b * m0, m0),
        ([init_feats[..., d].reshape(-1) for d in range(3)], idx0l, n0,
         b * m0, m0),
        ([xyz1[..., d].reshape(-1) for d in range(3)], idx1l, m0, b * m1, m1),
    ])
    gx0 = gx0.reshape(_K, b * m0, 16)
    gf0 = gf0.reshape(_K, b * m0, 16)
    wnw0p = jnp.concatenate(
        [wn0_W, jnp.zeros((13, 16), dtype=jnp.float32)], axis=0
    )
    f1 = _tc_layer0(
        gx0, gf0, _pad16(xyz1.reshape(b * m0, 3)), wnw0p, wn0_b,
        lin0_W.reshape(3, 16, 256), lin0_b, mt=512,
    )

    idx1 = _flat_idx(nei_inds1, m0)
    (gf1,) = _sc_gather([f1], idx1)
    gx1 = gx1.reshape(_K, b * m1, 16)
    gf1 = gf1.reshape(_K, b * m1, 256)
    wnw1p = jnp.concatenate(
        [wn1_W, jnp.zeros((13, 16), dtype=jnp.float32)], axis=0
    )
    w3_bf = _wmajor(lin1_W, 256).astype(jnp.bfloat16).reshape(_W, 256, 1024)
    out = _tc_layer1(
        gx1, gf1, _pad16(xyz2.reshape(b * m1, 3)), wnw1p, wn1_b,
        w3_bf, lin1_b, mt=256,
    )
    return out.reshape(b, m1, 1024)
